# baseline (speedup 1.0000x reference)
import jax
import jax.numpy as jnp
from jax import lax
from jax.experimental import pallas as pl
from jax.experimental.pallas import tpu as pltpu
from jax.experimental.pallas import tpu_sc as plsc

N_NODES = 10000
N_EDGES = 160000
IN_CH = 256
HIDDEN = 64
OUT_CH = 128

NC, NS = 2, 16
NW = NC * NS
K = 40
EPT = 5000
NCH = EPT // K
NP = 10240
RPT = NP // NS
HT = NP // 128
R_BLK = 1024


def _sc_mesh():
    return plsc.VectorSubcoreMesh(core_axis_name="c", subcore_axis_name="s")


def _sc_degree():
    def body(dst_hbm, zeros_hbm, out_hbm, dst_v, hist_v):
        c = lax.axis_index("c")
        s = lax.axis_index("s")
        w = c * NS + s
        nfull = EPT // 16
        rem = EPT - nfull * 16
        if rem:
            dst_v[pl.ds(nfull * 16, 16)] = jnp.zeros((16,), jnp.int32)
        pltpu.sync_copy(zeros_hbm, hist_v)
        pltpu.sync_copy(dst_hbm.at[pl.ds(w * EPT, EPT)], dst_v.at[pl.ds(0, EPT)])
        ones16 = jnp.ones((16,), jnp.float32)

        def scatter_ones(idx, mask=None):
            row = lax.shift_right_logical(idx, 7)
            col = lax.bitwise_and(idx, 127)
            plsc.addupdate_scatter(hist_v, [row, col], ones16, mask=mask)

        def step(i, carry):
            scatter_ones(dst_v[pl.ds(i * 16, 16)])
            return carry

        lax.fori_loop(0, nfull, step, 0)
        if rem:
            tail_mask = lax.iota(jnp.int32, 16) < rem
            scatter_ones(dst_v[pl.ds(nfull * 16, 16)], tail_mask)
        pltpu.sync_copy(hist_v, out_hbm.at[w])

    return pl.kernel(
        body,
        out_type=jax.ShapeDtypeStruct((NW, HT, 128), jnp.float32),
        mesh=_sc_mesh(),
        compiler_params=pltpu.CompilerParams(needs_layout_passes=False),
        scratch_types=[
            pltpu.VMEM((((EPT + 15) // 16) * 16,), jnp.int32),
            pltpu.VMEM((HT, 128), jnp.float32),
        ],
    )


NBUF = 4
LEAD = 2


def _sc_aggregate(ch):
    def body(xw_hbm, src_hbm, dst_hbm, zeros_hbm, out_hbm,
             src_v, dst_v, bufs, shared, gsems, ssems):
        c = lax.axis_index("c")
        s = lax.axis_index("s")
        w = c * NS + s

        @pl.when(c == 0)
        def _():
            pltpu.sync_copy(xw_hbm.at[pl.ds(s * RPT, RPT)],
                            shared.at[pl.ds(s * RPT, RPT)])

        @pl.when(c != 0)
        def _():
            pltpu.sync_copy(zeros_hbm, shared.at[pl.ds(s * RPT, RPT)])

        pltpu.sync_copy(src_hbm.at[pl.ds(w * EPT, EPT)], src_v)
        pltpu.sync_copy(dst_hbm.at[w], dst_v)
        plsc.subcore_barrier()

        def gather(f):
            return pltpu.async_copy(xw_hbm.at[src_v.at[pl.ds(f * K, K)]],
                                    bufs[f % NBUF], gsems[f % NBUF])

        def scat(j):
            return pltpu.async_copy(bufs[j % NBUF], shared.at[dst_v.at[j]],
                                    ssems[j % NBUF], add=True)

        gd, sd = {}, {}
        for f in range(LEAD):
            gd[f] = gather(f)
        for j in range(NCH):
            gd[j].wait()
            sd[j] = scat(j)
            f = j + LEAD
            if f < NCH:
                if f >= NBUF:
                    sd[f - NBUF].wait()
                gd[f] = gather(f)
        for j in range(NCH - NBUF, NCH):
            sd[j].wait()
        plsc.subcore_barrier()
        pltpu.sync_copy(shared.at[pl.ds(s * RPT, RPT)],
                        out_hbm.at[c, pl.ds(s * RPT, RPT)])

    return pl.kernel(
        body,
        out_type=jax.ShapeDtypeStruct((NC, NP, ch), jnp.float32),
        mesh=_sc_mesh(),
        scratch_types=[
            pltpu.VMEM((EPT,), jnp.int32),
            pltpu.VMEM((NCH, K), jnp.int32),
            [pltpu.VMEM((K, ch), jnp.float32) for _ in range(NBUF)],
            pltpu.VMEM_SHARED((NP, ch), jnp.float32),
            [pltpu.SemaphoreType.DMA for _ in range(NBUF)],
            [pltpu.SemaphoreType.DMA for _ in range(NBUF)],
        ],
    )


_RT = R_BLK // 128


def _row_scale(m, dinv_t):
    parts = []
    for r in range(_RT):
        parts.append(m[128 * r:128 * (r + 1), :] * dinv_t[:, r:r + 1])
    return jnp.concatenate(parts, axis=0)


def _t1_body(hist_ref, x_ref, w1_ref, dinv_ref, xwp_ref):
    deg = jnp.sum(hist_ref[...], axis=0) + 1.0
    dinv = lax.rsqrt(deg)
    dinv_ref[...] = dinv
    dinv_t = jnp.transpose(dinv)
    xw = jnp.dot(x_ref[...], w1_ref[...], preferred_element_type=jnp.float32)
    xwp_ref[:, :HIDDEN] = _row_scale(xw, dinv_t)
    xwp_ref[:, HIDDEN:] = jnp.zeros((R_BLK, OUT_CH - HIDDEN), jnp.float32)


def _t1(hist, x, W1):
    return pl.pallas_call(
        _t1_body,
        grid=(NP // R_BLK,),
        in_specs=[
            pl.BlockSpec((NW, _RT, 128), lambda i: (0, i, 0)),
            pl.BlockSpec((R_BLK, IN_CH), lambda i: (i, 0)),
            pl.BlockSpec((IN_CH, HIDDEN), lambda i: (0, 0)),
        ],
        out_specs=[
            pl.BlockSpec((_RT, 128), lambda i: (i, 0)),
            pl.BlockSpec((R_BLK, OUT_CH), lambda i: (i, 0)),
        ],
        out_shape=[
            jax.ShapeDtypeStruct((NP // 128, 128), jnp.float32),
            jax.ShapeDtypeStruct((NP, OUT_CH), jnp.float32),
        ],
    )(hist, x, W1)


def _t2_body(a_ref, dinv_ref, b1_ref, w2_ref, out_ref):
    dinv_t = jnp.transpose(dinv_ref[...])
    agg = a_ref[0, :, :HIDDEN] + a_ref[1, :, :HIDDEN]
    h = jnp.maximum(_row_scale(agg, dinv_t) + b1_ref[...], 0.0)
    hw = jnp.dot(h, w2_ref[...], preferred_element_type=jnp.float32)
    out_ref[...] = _row_scale(hw, dinv_t)


def _t2(a1, dinv, b1, W2):
    return pl.pallas_call(
        _t2_body,
        grid=(NP // R_BLK,),
        in_specs=[
            pl.BlockSpec((NC, R_BLK, OUT_CH), lambda i: (0, i, 0)),
            pl.BlockSpec((_RT, 128), lambda i: (i, 0)),
            pl.BlockSpec((1, HIDDEN), lambda i: (0, 0)),
            pl.BlockSpec((HIDDEN, OUT_CH), lambda i: (0, 0)),
        ],
        out_specs=pl.BlockSpec((R_BLK, OUT_CH), lambda i: (i, 0)),
        out_shape=jax.ShapeDtypeStruct((NP, OUT_CH), jnp.float32),
    )(a1, dinv, b1, W2)


def _t3_body(a_ref, dinv_ref, b2_ref, out_ref):
    dinv_t = jnp.transpose(dinv_ref[...])
    acc = a_ref[0] + a_ref[1]
    out_ref[...] = _row_scale(acc, dinv_t) + b2_ref[...]


def _t3(a2, dinv, b2):
    return pl.pallas_call(
        _t3_body,
        grid=(NP // R_BLK,),
        in_specs=[
            pl.BlockSpec((NC, R_BLK, OUT_CH), lambda i: (0, i, 0)),
            pl.BlockSpec((_RT, 128), lambda i: (i, 0)),
            pl.BlockSpec((1, OUT_CH), lambda i: (0, 0)),
        ],
        out_specs=pl.BlockSpec((R_BLK, OUT_CH), lambda i: (i, 0)),
        out_shape=jax.ShapeDtypeStruct((N_NODES, OUT_CH), jnp.float32),
    )(a2, dinv, b2)


def kernel(x, edge_index, W1, b1, W2, b2):
    ei = edge_index.astype(jnp.int32)
    src = ei[0]
    dst = ei[1]
    dst3 = dst.reshape(NW, NCH, K)
    zh = jnp.zeros((HT, 128), jnp.float32)
    z128 = jnp.zeros((RPT, OUT_CH), jnp.float32)

    hist = _sc_degree()(dst, zh)
    dinv, xwp = _t1(hist, x, W1)
    a1 = _sc_aggregate(OUT_CH)(xwp, src, dst3, z128)
    hwp = _t2(a1, dinv, b1.reshape(1, HIDDEN), W2)
    a2 = _sc_aggregate(OUT_CH)(hwp, src, dst3, z128)
    return _t3(a2, dinv, b2.reshape(1, OUT_CH))


# speedup vs baseline: 1.0657x; 1.0657x over previous
import jax
import jax.numpy as jnp
from jax import lax
from jax.experimental import pallas as pl
from jax.experimental.pallas import tpu as pltpu
from jax.experimental.pallas import tpu_sc as plsc

N_NODES = 10000
N_EDGES = 160000
IN_CH = 256
HIDDEN = 64
OUT_CH = 128

NC, NS = 2, 16
NW = NC * NS
K = 64
EPT = 5120
NCH = EPT // K
EP = NW * EPT
NP = 10240
RPT = NP // NS
HT = NP // 128
R_BLK = 1024


def _sc_mesh():
    return plsc.VectorSubcoreMesh(core_axis_name="c", subcore_axis_name="s")


def _sc_degree():
    def body(dst_hbm, zeros_hbm, out_hbm, dst_v, hist_v):
        c = lax.axis_index("c")
        s = lax.axis_index("s")
        w = c * NS + s
        nfull = EPT // 16
        rem = EPT - nfull * 16
        if rem:
            dst_v[pl.ds(nfull * 16, 16)] = jnp.zeros((16,), jnp.int32)
        pltpu.sync_copy(zeros_hbm, hist_v)
        pltpu.sync_copy(dst_hbm.at[pl.ds(w * EPT, EPT)], dst_v.at[pl.ds(0, EPT)])
        ones16 = jnp.ones((16,), jnp.float32)

        def scatter_ones(idx, mask=None):
            row = lax.shift_right_logical(idx, 7)
            col = lax.bitwise_and(idx, 127)
            plsc.addupdate_scatter(hist_v, [row, col], ones16, mask=mask)

        def step(i, carry):
            scatter_ones(dst_v[pl.ds(i * 16, 16)])
            return carry

        lax.fori_loop(0, nfull, step, 0)
        if rem:
            tail_mask = lax.iota(jnp.int32, 16) < rem
            scatter_ones(dst_v[pl.ds(nfull * 16, 16)], tail_mask)
        pltpu.sync_copy(hist_v, out_hbm.at[w])

    return pl.kernel(
        body,
        out_type=jax.ShapeDtypeStruct((NW, HT, 128), jnp.float32),
        mesh=_sc_mesh(),
        compiler_params=pltpu.CompilerParams(needs_layout_passes=False),
        scratch_types=[
            pltpu.VMEM((((EPT + 15) // 16) * 16,), jnp.int32),
            pltpu.VMEM((HT, 128), jnp.float32),
        ],
    )


NBUF = 4
LEAD = 2


def _sc_aggregate(ch):
    def body(xw_hbm, src_hbm, dst_hbm, zeros_hbm, out_hbm,
             src_v, dst_v, bufs, shared, gsems, ssems):
        c = lax.axis_index("c")
        s = lax.axis_index("s")
        w = c * NS + s

        @pl.when(c == 0)
        def _():
            pltpu.sync_copy(xw_hbm.at[pl.ds(s * RPT, RPT)],
                            shared.at[pl.ds(s * RPT, RPT)])

        @pl.when(c != 0)
        def _():
            pltpu.sync_copy(zeros_hbm, shared.at[pl.ds(s * RPT, RPT)])

        pltpu.sync_copy(src_hbm.at[pl.ds(w * EPT, EPT)], src_v)
        pltpu.sync_copy(dst_hbm.at[w], dst_v)
        plsc.subcore_barrier()

        def gather(f):
            return pltpu.async_copy(xw_hbm.at[src_v.at[pl.ds(f * K, K)]],
                                    bufs[f % NBUF], gsems[f % NBUF])

        def scat(j):
            return pltpu.async_copy(bufs[j % NBUF], shared.at[dst_v.at[j]],
                                    ssems[j % NBUF], add=True)

        gd, sd = {}, {}
        for f in range(LEAD):
            gd[f] = gather(f)
        for j in range(NCH):
            gd[j].wait()
            sd[j] = scat(j)
            f = j + LEAD
            if f < NCH:
                if f >= NBUF:
                    sd[f - NBUF].wait()
                gd[f] = gather(f)
        for j in range(NCH - NBUF, NCH):
            sd[j].wait()
        plsc.subcore_barrier()
        pltpu.sync_copy(shared.at[pl.ds(s * RPT, RPT)],
                        out_hbm.at[c, pl.ds(s * RPT, RPT)])

    return pl.kernel(
        body,
        out_type=jax.ShapeDtypeStruct((NC, NP, ch), jnp.float32),
        mesh=_sc_mesh(),
        scratch_types=[
            pltpu.VMEM((EPT,), jnp.int32),
            pltpu.VMEM((NCH, K), jnp.int32),
            [pltpu.VMEM((K, ch), jnp.float32) for _ in range(NBUF)],
            pltpu.VMEM_SHARED((NP, ch), jnp.float32),
            [pltpu.SemaphoreType.DMA for _ in range(NBUF)],
            [pltpu.SemaphoreType.DMA for _ in range(NBUF)],
        ],
    )


_RT = R_BLK // 128


def _row_scale(m, dinv_t):
    parts = []
    for r in range(_RT):
        parts.append(m[128 * r:128 * (r + 1), :] * dinv_t[:, r:r + 1])
    return jnp.concatenate(parts, axis=0)


def _t1_body(hist_ref, x_ref, w1_ref, dinv_ref, xwp_ref):
    deg = jnp.sum(hist_ref[...], axis=0) + 1.0
    dinv = lax.rsqrt(deg)
    dinv_ref[...] = dinv
    dinv_t = jnp.transpose(dinv)
    xw = jnp.dot(x_ref[...], w1_ref[...], preferred_element_type=jnp.float32)
    xwp_ref[:, :HIDDEN] = _row_scale(xw, dinv_t)
    xwp_ref[:, HIDDEN:] = jnp.zeros((R_BLK, OUT_CH - HIDDEN), jnp.float32)


def _t1(hist, x, W1):
    return pl.pallas_call(
        _t1_body,
        grid=(NP // R_BLK,),
        in_specs=[
            pl.BlockSpec((NW, _RT, 128), lambda i: (0, i, 0)),
            pl.BlockSpec((R_BLK, IN_CH), lambda i: (i, 0)),
            pl.BlockSpec((IN_CH, HIDDEN), lambda i: (0, 0)),
        ],
        out_specs=[
            pl.BlockSpec((_RT, 128), lambda i: (i, 0)),
            pl.BlockSpec((R_BLK, OUT_CH), lambda i: (i, 0)),
        ],
        out_shape=[
            jax.ShapeDtypeStruct((NP // 128, 128), jnp.float32),
            jax.ShapeDtypeStruct((NP, OUT_CH), jnp.float32),
        ],
    )(hist, x, W1)


def _t2_body(a_ref, dinv_ref, b1_ref, w2_ref, out_ref):
    dinv_t = jnp.transpose(dinv_ref[...])
    agg = a_ref[0, :, :HIDDEN] + a_ref[1, :, :HIDDEN]
    h = jnp.maximum(_row_scale(agg, dinv_t) + b1_ref[...], 0.0)
    hw = jnp.dot(h, w2_ref[...], preferred_element_type=jnp.float32)
    out_ref[...] = _row_scale(hw, dinv_t)


def _t2(a1, dinv, b1, W2):
    return pl.pallas_call(
        _t2_body,
        grid=(NP // R_BLK,),
        in_specs=[
            pl.BlockSpec((NC, R_BLK, OUT_CH), lambda i: (0, i, 0)),
            pl.BlockSpec((_RT, 128), lambda i: (i, 0)),
            pl.BlockSpec((1, HIDDEN), lambda i: (0, 0)),
            pl.BlockSpec((HIDDEN, OUT_CH), lambda i: (0, 0)),
        ],
        out_specs=pl.BlockSpec((R_BLK, OUT_CH), lambda i: (i, 0)),
        out_shape=jax.ShapeDtypeStruct((NP, OUT_CH), jnp.float32),
    )(a1, dinv, b1, W2)


def _t3_body(a_ref, dinv_ref, b2_ref, out_ref):
    dinv_t = jnp.transpose(dinv_ref[...])
    acc = a_ref[0] + a_ref[1]
    out_ref[...] = _row_scale(acc, dinv_t) + b2_ref[...]


def _t3(a2, dinv, b2):
    return pl.pallas_call(
        _t3_body,
        grid=(NP // R_BLK,),
        in_specs=[
            pl.BlockSpec((NC, R_BLK, OUT_CH), lambda i: (0, i, 0)),
            pl.BlockSpec((_RT, 128), lambda i: (i, 0)),
            pl.BlockSpec((1, OUT_CH), lambda i: (0, 0)),
        ],
        out_specs=pl.BlockSpec((R_BLK, OUT_CH), lambda i: (i, 0)),
        out_shape=jax.ShapeDtypeStruct((N_NODES, OUT_CH), jnp.float32),
    )(a2, dinv, b2)


def kernel(x, edge_index, W1, b1, W2, b2):
    ei = edge_index.astype(jnp.int32)
    npad = EP - N_EDGES
    pad_rows = N_NODES + (jnp.arange(npad, dtype=jnp.int32) % (NP - N_NODES))
    src = jnp.concatenate([ei[0], pad_rows])
    dst = jnp.concatenate([ei[1], pad_rows])
    dst3 = dst.reshape(NW, NCH, K)
    zh = jnp.zeros((HT, 128), jnp.float32)
    z128 = jnp.zeros((RPT, OUT_CH), jnp.float32)

    hist = _sc_degree()(dst, zh)
    dinv, xwp = _t1(hist, x, W1)
    a1 = _sc_aggregate(OUT_CH)(xwp, src, dst3, z128)
    hwp = _t2(a1, dinv, b1.reshape(1, HIDDEN), W2)
    a2 = _sc_aggregate(OUT_CH)(hwp, src, dst3, z128)
    return _t3(a2, dinv, b2.reshape(1, OUT_CH))


# speedup vs baseline: 1.0738x; 1.0076x over previous
import jax
import jax.numpy as jnp
import numpy as np
from jax import lax
from jax.experimental import pallas as pl
from jax.experimental.pallas import tpu as pltpu
from jax.experimental.pallas import tpu_sc as plsc

N_NODES = 10000
N_EDGES = 160000
IN_CH = 256
HIDDEN = 64
OUT_CH = 128

NC, NS = 2, 16
NW = NC * NS
K = 64
EPT = 5120
NCH = EPT // K
EP = NW * EPT
NP = 10240
RPT = NP // NS
HT = NP // 128
R_BLK = 1024


def _sc_mesh():
    return plsc.VectorSubcoreMesh(core_axis_name="c", subcore_axis_name="s")


EPT_D = N_EDGES // NW


def _sc_degree():
    def body(dst_hbm, zeros_hbm, out_hbm, dst_v, hist_v):
        c = lax.axis_index("c")
        s = lax.axis_index("s")
        w = c * NS + s
        nfull = EPT_D // 16
        rem = EPT_D - nfull * 16
        if rem:
            dst_v[pl.ds(nfull * 16, 16)] = jnp.zeros((16,), jnp.int32)
        pltpu.sync_copy(zeros_hbm, hist_v)
        pltpu.sync_copy(dst_hbm.at[pl.ds(w * EPT_D, EPT_D)],
                        dst_v.at[pl.ds(0, EPT_D)])
        ones16 = jnp.ones((16,), jnp.float32)

        def scatter_ones(idx, mask=None):
            row = lax.shift_right_logical(idx, 7)
            col = lax.bitwise_and(idx, 127)
            plsc.addupdate_scatter(hist_v, [row, col], ones16, mask=mask)

        def step(i, carry):
            scatter_ones(dst_v[pl.ds(i * 16, 16)])
            return carry

        lax.fori_loop(0, nfull, step, 0)
        if rem:
            tail_mask = lax.iota(jnp.int32, 16) < rem
            scatter_ones(dst_v[pl.ds(nfull * 16, 16)], tail_mask)
        pltpu.sync_copy(hist_v, out_hbm.at[w])

    return pl.kernel(
        body,
        out_type=jax.ShapeDtypeStruct((NW, HT, 128), jnp.float32),
        mesh=_sc_mesh(),
        compiler_params=pltpu.CompilerParams(needs_layout_passes=False),
        scratch_types=[
            pltpu.VMEM((((EPT_D + 15) // 16) * 16,), jnp.int32),
            pltpu.VMEM((HT, 128), jnp.float32),
        ],
    )


NBUF = 4
LEAD = 2


def _sc_aggregate(ch):
    def body(xw_hbm, src_hbm, dst_hbm, zeros_hbm, out_hbm,
             src_v, dst_v, bufs, shared, gsems, ssems):
        c = lax.axis_index("c")
        s = lax.axis_index("s")
        w = c * NS + s

        @pl.when(c == 0)
        def _():
            pltpu.sync_copy(xw_hbm.at[pl.ds(s * RPT, RPT)],
                            shared.at[pl.ds(s * RPT, RPT)])

        @pl.when(c != 0)
        def _():
            pltpu.sync_copy(zeros_hbm, shared.at[pl.ds(s * RPT, RPT)])

        pltpu.sync_copy(src_hbm.at[pl.ds(w * EPT, EPT)], src_v)
        pltpu.sync_copy(dst_hbm.at[w], dst_v)
        plsc.subcore_barrier()

        def gather(f):
            return pltpu.async_copy(xw_hbm.at[src_v.at[pl.ds(f * K, K)]],
                                    bufs[f % NBUF], gsems[f % NBUF])

        def scat(j):
            return pltpu.async_copy(bufs[j % NBUF], shared.at[dst_v.at[j]],
                                    ssems[j % NBUF], add=True)

        gd, sd = {}, {}
        for f in range(LEAD):
            gd[f] = gather(f)
        for j in range(NCH):
            gd[j].wait()
            sd[j] = scat(j)
            f = j + LEAD
            if f < NCH:
                if f >= NBUF:
                    sd[f - NBUF].wait()
                gd[f] = gather(f)
        for j in range(NCH - NBUF, NCH):
            sd[j].wait()
        plsc.subcore_barrier()
        pltpu.sync_copy(shared.at[pl.ds(s * RPT, RPT)],
                        out_hbm.at[c, pl.ds(s * RPT, RPT)])

    return pl.kernel(
        body,
        out_type=jax.ShapeDtypeStruct((NC, NP, ch), jnp.float32),
        mesh=_sc_mesh(),
        scratch_types=[
            pltpu.VMEM((EPT,), jnp.int32),
            pltpu.VMEM((NCH, K), jnp.int32),
            [pltpu.VMEM((K, ch), jnp.float32) for _ in range(NBUF)],
            pltpu.VMEM_SHARED((NP, ch), jnp.float32),
            [pltpu.SemaphoreType.DMA for _ in range(NBUF)],
            [pltpu.SemaphoreType.DMA for _ in range(NBUF)],
        ],
    )


_RT = R_BLK // 128


def _row_scale(m, dinv_t):
    parts = []
    for r in range(_RT):
        parts.append(m[128 * r:128 * (r + 1), :] * dinv_t[:, r:r + 1])
    return jnp.concatenate(parts, axis=0)


def _t1_body(hist_ref, x_ref, w1_ref, dinv_ref, xwp_ref):
    deg = jnp.sum(hist_ref[...], axis=0) + 1.0
    dinv = lax.rsqrt(deg)
    dinv_ref[...] = dinv
    dinv_t = jnp.transpose(dinv)
    xw = jnp.dot(x_ref[...], w1_ref[...], preferred_element_type=jnp.float32)
    xwp_ref[:, :HIDDEN] = _row_scale(xw, dinv_t)
    xwp_ref[:, HIDDEN:] = jnp.zeros((R_BLK, OUT_CH - HIDDEN), jnp.float32)


def _t1(hist, x, W1):
    return pl.pallas_call(
        _t1_body,
        grid=(NP // R_BLK,),
        in_specs=[
            pl.BlockSpec((NW, _RT, 128), lambda i: (0, i, 0)),
            pl.BlockSpec((R_BLK, IN_CH), lambda i: (i, 0)),
            pl.BlockSpec((IN_CH, HIDDEN), lambda i: (0, 0)),
        ],
        out_specs=[
            pl.BlockSpec((_RT, 128), lambda i: (i, 0)),
            pl.BlockSpec((R_BLK, OUT_CH), lambda i: (i, 0)),
        ],
        out_shape=[
            jax.ShapeDtypeStruct((NP // 128, 128), jnp.float32),
            jax.ShapeDtypeStruct((NP, OUT_CH), jnp.float32),
        ],
    )(hist, x, W1)


def _t2_body(a_ref, dinv_ref, b1_ref, w2_ref, out_ref):
    dinv_t = jnp.transpose(dinv_ref[...])
    agg = a_ref[0, :, :HIDDEN] + a_ref[1, :, :HIDDEN]
    h = jnp.maximum(_row_scale(agg, dinv_t) + b1_ref[...], 0.0)
    hw = jnp.dot(h, w2_ref[...], preferred_element_type=jnp.float32)
    out_ref[...] = _row_scale(hw, dinv_t)


def _t2(a1, dinv, b1, W2):
    return pl.pallas_call(
        _t2_body,
        grid=(NP // R_BLK,),
        in_specs=[
            pl.BlockSpec((NC, R_BLK, OUT_CH), lambda i: (0, i, 0)),
            pl.BlockSpec((_RT, 128), lambda i: (i, 0)),
            pl.BlockSpec((1, HIDDEN), lambda i: (0, 0)),
            pl.BlockSpec((HIDDEN, OUT_CH), lambda i: (0, 0)),
        ],
        out_specs=pl.BlockSpec((R_BLK, OUT_CH), lambda i: (i, 0)),
        out_shape=jax.ShapeDtypeStruct((NP, OUT_CH), jnp.float32),
    )(a1, dinv, b1, W2)


def _t3_body(a_ref, dinv_ref, b2_ref, out_ref):
    dinv_t = jnp.transpose(dinv_ref[...])
    acc = a_ref[0] + a_ref[1]
    out_ref[...] = _row_scale(acc, dinv_t) + b2_ref[...]


def _t3(a2, dinv, b2):
    return pl.pallas_call(
        _t3_body,
        grid=(NP // R_BLK,),
        in_specs=[
            pl.BlockSpec((NC, R_BLK, OUT_CH), lambda i: (0, i, 0)),
            pl.BlockSpec((_RT, 128), lambda i: (i, 0)),
            pl.BlockSpec((1, OUT_CH), lambda i: (0, 0)),
        ],
        out_specs=pl.BlockSpec((R_BLK, OUT_CH), lambda i: (i, 0)),
        out_shape=jax.ShapeDtypeStruct((N_NODES, OUT_CH), jnp.float32),
    )(a2, dinv, b2)


def kernel(x, edge_index, W1, b1, W2, b2):
    ei = edge_index.astype(jnp.int32)
    npad = EP - N_EDGES
    pad_rows = jnp.asarray(
        N_NODES + (np.arange(npad) % (NP - N_NODES)), dtype=jnp.int32)
    src = jnp.concatenate([ei[0], pad_rows])
    dst = jnp.concatenate([ei[1], pad_rows])
    dst3 = dst.reshape(NW, NCH, K)
    zh = jnp.zeros((HT, 128), jnp.float32)
    z128 = jnp.zeros((RPT, OUT_CH), jnp.float32)

    hist = _sc_degree()(ei[1], zh)
    dinv, xwp = _t1(hist, x, W1)
    a1 = _sc_aggregate(OUT_CH)(xwp, src, dst3, z128)
    hwp = _t2(a1, dinv, b1.reshape(1, HIDDEN), W2)
    a2 = _sc_aggregate(OUT_CH)(hwp, src, dst3, z128)
    return _t3(a2, dinv, b2.reshape(1, OUT_CH))
